# baseline (speedup 1.0000x reference)
import functools

import jax
import jax.numpy as jnp
from jax import lax
from jax.experimental import pallas as pl
from jax.experimental.pallas import tpu as pltpu
from jax.experimental.pallas import tpu_sc as plsc

BATCH = 512
D = 64
K = 4
NC = 2
NS = 16
NW = NC * NS
BLK = 98304


def _tc_project_table(fc_w, emb_t, c):
  n = emb_t.shape[1]
  grid = (n + BLK - 1) // BLK
  inv_c = float(1.0 / c)

  def body(w_ref, e_ref, *o_refs):
    pt = jnp.dot(w_ref[...] * inv_c, e_ref[...],
                 preferred_element_type=jnp.float32)
    for kk in range(K):
      o_refs[kk][...] = pt[kk]

  return pl.pallas_call(
      body,
      grid=(grid,),
      in_specs=[
          pl.BlockSpec((K, D), lambda i: (0, 0)),
          pl.BlockSpec((D, BLK), lambda i: (0, i)),
      ],
      out_specs=[pl.BlockSpec((BLK,), lambda i: (i,)) for _ in range(K)],
      out_shape=[jax.ShapeDtypeStruct((n,), jnp.float32) for _ in range(K)],
  )(fc_w, emb_t)


def _sc_gather_segsum(text, pt, c):
  sw = BATCH // NW
  tw = sw * c
  assert c % 8 == 0 and sw % 2 == 0 and tw % 128 == 0
  npair = 2 * c // 16
  straddle = c % 16 != 0
  mesh = plsc.VectorSubcoreMesh(core_axis_name="c", subcore_axis_name="s")

  @functools.partial(
      pl.kernel,
      mesh=mesh,
      compiler_params=pltpu.CompilerParams(
          needs_layout_passes=False, use_tc_tiling_on_sc=False),
      out_type=jax.ShapeDtypeStruct((K, BATCH), jnp.float32),
      scratch_types=[
          pltpu.VMEM((tw,), jnp.int32),
          pltpu.VMEM((K, tw), jnp.float32),
          pltpu.VMEM((K, 16), jnp.float32),
          pltpu.SemaphoreType.DMA,
      ],
  )
  def k(text_hbm, p0, p1, p2, p3, out, idx_v, d_v, acc_v, sem):
    wid = lax.axis_index("s") * NC + lax.axis_index("c")
    tbase = wid * tw
    pltpu.sync_copy(text_hbm.at[pl.ds(tbase, tw)], idx_v)
    ps = (p0, p1, p2, p3)
    for kk in range(K):
      pltpu.async_copy(ps[kk].at[idx_v], d_v.at[kk], sem)
    for kk in range(K):
      pltpu.make_async_copy(ps[kk].at[idx_v], d_v.at[kk], sem).wait()

    iota = lax.iota(jnp.int32, 16)
    one = jnp.full((16,), 1, jnp.int32)
    zero = jnp.full((16,), 0, jnp.int32)
    mask_lo = jnp.minimum(jnp.maximum(jnp.full((16,), 8, jnp.int32) - iota,
                                      zero), one).astype(jnp.float32)
    mask_hi = jnp.float32(1.0) - mask_lo

    def onehot(pos):
      dlt = iota - jnp.full((16,), pos, jnp.int32)
      return (one - jnp.minimum(dlt * dlt, one)).astype(jnp.float32)

    mid = c // 16

    def pair_body(p, accs):
      accs = list(accs)
      base = p * 2 * c
      for kk in range(K):
        psum_a = jnp.zeros((16,), jnp.float32)
        psum_b = jnp.zeros((16,), jnp.float32)
        for j in range(mid):
          psum_a = psum_a + d_v[kk, pl.ds(base + j * 16, 16)]
        if straddle:
          w = d_v[kk, pl.ds(base + mid * 16, 16)]
          psum_a = psum_a + w * mask_lo
          psum_b = psum_b + w * mask_hi
        for j in range(mid + (1 if straddle else 0), npair):
          psum_b = psum_b + d_v[kk, pl.ds(base + j * 16, 16)]
        ta = jnp.full((16,), jnp.sum(psum_a), jnp.float32)
        tb = jnp.full((16,), jnp.sum(psum_b), jnp.float32)
        accs[kk] = accs[kk] + ta * onehot(2 * p) + tb * onehot(2 * p + 1)
      return tuple(accs)

    accs = lax.fori_loop(
        0, sw // 2, pair_body,
        tuple(jnp.zeros((16,), jnp.float32) for _ in range(K)))
    for kk in range(K):
      acc_v[kk, pl.ds(0, 16)] = accs[kk]
      pltpu.sync_copy(acc_v.at[kk], out.at[kk].at[pl.ds(wid * sw, sw)])

  return k(text, *pt)


def kernel(text, emb_table, fc_w, fc_b):
  n = text.shape[0]
  c = n // BATCH
  assert BATCH * c == n and c % 8 == 0 and emb_table.shape[1] == D
  pt = _tc_project_table(fc_w.astype(jnp.float32), emb_table.T, c)
  out = _sc_gather_segsum(text.astype(jnp.int32), pt, c)
  return out.T + fc_b.astype(jnp.float32)


# speedup vs baseline: 1.0126x; 1.0126x over previous
import functools

import jax
import jax.numpy as jnp
from jax import lax
from jax.experimental import pallas as pl
from jax.experimental.pallas import tpu as pltpu
from jax.experimental.pallas import tpu_sc as plsc

BATCH = 512
D = 64
K = 4
NC = 2
NS = 16
NW = NC * NS
BLK = 32768


def _tc_project_table(fc_w, emb_t, c):
  n = emb_t.shape[1]
  grid = (n + BLK - 1) // BLK
  inv_c = float(1.0 / c)

  def body(w_ref, e_ref, *o_refs):
    pt = jnp.dot(w_ref[...] * inv_c, e_ref[...],
                 preferred_element_type=jnp.float32)
    for kk in range(K):
      o_refs[kk][...] = pt[kk]

  return pl.pallas_call(
      body,
      grid=(grid,),
      in_specs=[
          pl.BlockSpec((K, D), lambda i: (0, 0)),
          pl.BlockSpec((D, BLK), lambda i: (0, i)),
      ],
      out_specs=[pl.BlockSpec((BLK,), lambda i: (i,)) for _ in range(K)],
      out_shape=[jax.ShapeDtypeStruct((n,), jnp.float32) for _ in range(K)],
  )(fc_w, emb_t)


def _sc_gather_segsum(text, pt, c):
  sw = BATCH // NW
  tw = sw * c
  assert c % 8 == 0 and sw % 2 == 0 and tw % 128 == 0
  npair = 2 * c // 16
  straddle = c % 16 != 0
  mesh = plsc.VectorSubcoreMesh(core_axis_name="c", subcore_axis_name="s")

  @functools.partial(
      pl.kernel,
      mesh=mesh,
      compiler_params=pltpu.CompilerParams(
          needs_layout_passes=False, use_tc_tiling_on_sc=False),
      out_type=jax.ShapeDtypeStruct((K, BATCH), jnp.float32),
      scratch_types=[
          pltpu.VMEM((tw,), jnp.int32),
          pltpu.VMEM((K, tw), jnp.float32),
          pltpu.VMEM((K, 16), jnp.float32),
          pltpu.SemaphoreType.DMA,
      ],
  )
  def k(text_hbm, p0, p1, p2, p3, out, idx_v, d_v, acc_v, sem):
    wid = lax.axis_index("s") * NC + lax.axis_index("c")
    tbase = wid * tw
    pltpu.sync_copy(text_hbm.at[pl.ds(tbase, tw)], idx_v)
    ps = (p0, p1, p2, p3)
    for kk in range(K):
      pltpu.async_copy(ps[kk].at[idx_v], d_v.at[kk], sem)
    for kk in range(K):
      pltpu.make_async_copy(ps[kk].at[idx_v], d_v.at[kk], sem).wait()

    iota = lax.iota(jnp.int32, 16)
    one = jnp.full((16,), 1, jnp.int32)
    zero = jnp.full((16,), 0, jnp.int32)
    mask_lo = jnp.minimum(jnp.maximum(jnp.full((16,), 8, jnp.int32) - iota,
                                      zero), one).astype(jnp.float32)
    mask_hi = jnp.float32(1.0) - mask_lo

    def onehot(pos):
      dlt = iota - jnp.full((16,), pos, jnp.int32)
      return (one - jnp.minimum(dlt * dlt, one)).astype(jnp.float32)

    mid = c // 16

    def pair_body(p, accs):
      accs = list(accs)
      base = p * 2 * c
      for kk in range(K):
        psum_a = jnp.zeros((16,), jnp.float32)
        psum_b = jnp.zeros((16,), jnp.float32)
        for j in range(mid):
          psum_a = psum_a + d_v[kk, pl.ds(base + j * 16, 16)]
        if straddle:
          w = d_v[kk, pl.ds(base + mid * 16, 16)]
          psum_a = psum_a + w * mask_lo
          psum_b = psum_b + w * mask_hi
        for j in range(mid + (1 if straddle else 0), npair):
          psum_b = psum_b + d_v[kk, pl.ds(base + j * 16, 16)]
        ta = jnp.full((16,), jnp.sum(psum_a), jnp.float32)
        tb = jnp.full((16,), jnp.sum(psum_b), jnp.float32)
        accs[kk] = accs[kk] + ta * onehot(2 * p) + tb * onehot(2 * p + 1)
      return tuple(accs)

    accs = lax.fori_loop(
        0, sw // 2, pair_body,
        tuple(jnp.zeros((16,), jnp.float32) for _ in range(K)))
    for kk in range(K):
      acc_v[kk, pl.ds(0, 16)] = accs[kk]
      pltpu.sync_copy(acc_v.at[kk], out.at[kk].at[pl.ds(wid * sw, sw)])

  return k(text, *pt)


def kernel(text, emb_table, fc_w, fc_b):
  n = text.shape[0]
  c = n // BATCH
  assert BATCH * c == n and c % 8 == 0 and emb_table.shape[1] == D
  pt = _tc_project_table(fc_w.astype(jnp.float32), emb_table.T, c)
  out = _sc_gather_segsum(text.astype(jnp.int32), pt, c)
  return out.T + fc_b.astype(jnp.float32)
